# Initial kernel scaffold; baseline (speedup 1.0000x reference)
import jax
import jax.numpy as jnp
from jax.experimental import pallas as pl


def kernel(x, edge_index, edge_attr, batch, edge_batch, att, W1, b1, W2, b2):
    raise NotImplementedError("write your pallas kernel here")


# speedup vs baseline: 1.0810x; 1.0810x over previous
import jax
import jax.numpy as jnp
from jax.experimental import pallas as pl

_TEMP = 0.1
_EPS = 1e-16


def _mlp_body(ea_ref, W1_ref, b1_ref, W2_ref, b2_ref, pi_ref):
    ea = ea_ref[...]
    h = jnp.maximum(
        jnp.dot(ea, W1_ref[...], preferred_element_type=jnp.float32) + b1_ref[...],
        0.0,
    )
    pi_ref[...] = jnp.dot(h, W2_ref[...], preferred_element_type=jnp.float32) + b2_ref[...]


def kernel(x, edge_index, edge_attr, batch, edge_batch, att, W1, b1, W2, b2):
    E = edge_attr.shape[0]
    BLK = 8000
    grid = E // BLK
    pi = pl.pallas_call(
        _mlp_body,
        grid=(grid,),
        in_specs=[
            pl.BlockSpec((BLK, 2), lambda i: (i, 0)),
            pl.BlockSpec((2, 128), lambda i: (0, 0)),
            pl.BlockSpec((1, 128), lambda i: (0, 0)),
            pl.BlockSpec((128, 1), lambda i: (0, 0)),
            pl.BlockSpec((1, 1), lambda i: (0, 0)),
        ],
        out_specs=pl.BlockSpec((BLK, 1), lambda i: (i, 0)),
        out_shape=jax.ShapeDtypeStruct((E, 1), jnp.float32),
    )(edge_attr, W1, b1.reshape(1, 128), W2, b2.reshape(1, 1))

    logits = pi / _TEMP
    m = jnp.max(logits)
    e = jnp.exp(logits - m)
    denom = jnp.sum(e)
    score = (e / (denom + _EPS)).reshape(-1)
    score = jnp.maximum(score, 0.0)
    k = E // 2
    _, perm = jax.lax.top_k(score, k)
    ei = edge_index[:, perm]
    ea2 = edge_attr[perm]
    used = jnp.zeros((x.shape[0],), dtype=bool).at[ei.reshape(-1)].set(True)
    new_idx = jnp.cumsum(used.astype(jnp.int32)) - 1
    ei = new_idx[ei]
    return (x, ei, ea2, batch)
